# baseline (speedup 1.0000x reference)
import jax
import jax.numpy as jnp
import numpy as np
from jax.experimental import pallas as pl
from jax.experimental.pallas import tpu as pltpu

N, D, C0, F0 = 10000, 16, 1, 128
C1, F1 = 4, 16
C2, F2 = 32, 1
NUM_CLASS = 40
CF = C1 * F1
CFF = C1 * F1 * F1

BLOCK_N = 400
CHUNK = 4


def _softsign(v):
    return v / (1.0 + jnp.abs(v))


def _sgnroot_pair(a):
    m = a * jax.lax.rsqrt(jnp.maximum(jnp.abs(a), 1e-30))
    return m, jnp.abs(m)


def _rep_tile_mats():
    rep = np.zeros((CF, CFF), dtype=np.float32)
    tile = np.zeros((CF, CFF), dtype=np.float32)
    for c in range(C1):
        for xx in range(F1):
            for yy in range(F1):
                j = c * F1 * F1 + xx * F1 + yy
                rep[c * F1 + xx, j] = 1.0
                tile[c * F1 + yy, j] = 1.0
    return rep, tile


def _klgl_block(x_ref, nbr_ref, w1_ref, b1_ref, w2_ref, b2_ref, wc_ref,
                bc_ref, rep_ref, tile_ref, seg_ref, nsum_ref, out_ref,
                zx_scr, zn_scr):
    B = x_ref.shape[0]
    xb = x_ref[:]
    nb = nbr_ref[:]
    sb = jnp.sum(nb, axis=1)

    for c in range(B // CHUNK):
        lo = c * CHUNK
        xc = xb[lo:lo + CHUNK]
        sc = sb[lo:lo + CHUNK]
        a = xc[:, :, None] * sc[:, None, :] + sc[:, :, None] * xc[:, None, :]
        m, r = _sgnroot_pair(a)
        rs = jax.lax.dot_general(jnp.ones((CHUNK, 1, F0), jnp.float32),
                                 r, (((2,), (1,)), ((0,), (0,))),
                                 preferred_element_type=jnp.float32) + 1e-7
        v = jnp.concatenate([nb[lo:lo + CHUNK], xc[:, None, :]], axis=1)
        z = jax.lax.dot_general(v, m, (((2,), (1,)), ((0,), (0,))),
                                preferred_element_type=jnp.float32) / rs
        zn_scr[lo:lo + CHUNK] = z[:, :D, :]
        zx_scr[lo:lo + CHUNK] = z[:, D, :]

    x1 = _softsign(jnp.dot(zx_scr[:], w1_ref[:],
                           preferred_element_type=jnp.float32) + b1_ref[:])
    hn = _softsign(jnp.dot(zn_scr[:].reshape(B * D, F0), w1_ref[:],
                           preferred_element_type=jnp.float32) + b1_ref[:])
    s2 = jnp.dot(nsum_ref[:], hn, preferred_element_type=jnp.float32)

    xs = jnp.concatenate([x1, s2], axis=0)
    reps = jnp.dot(xs, rep_ref[:], preferred_element_type=jnp.float32)
    tiles = jnp.dot(xs, tile_ref[:], preferred_element_type=jnp.float32)
    a2 = reps[:B] * tiles[B:] + reps[B:] * tiles[:B]
    m2, r2 = _sgnroot_pair(a2)
    stk = jnp.concatenate([r2, m2 * tiles[:B]], axis=0)
    sums = jnp.dot(stk, seg_ref[:], preferred_element_type=jnp.float32)
    zx2 = sums[B:] / (sums[:B] + 1e-7)

    x2 = jnp.dot(zx2, w2_ref[:],
                 preferred_element_type=jnp.float32) + b2_ref[:]
    x2 = _softsign(x2)
    out_ref[:] = jnp.dot(x2, wc_ref[:],
                         preferred_element_type=jnp.float32) + bc_ref[:]


@jax.jit
def kernel(x, neighbor, W1, b1, g1, be1, W2, b2, g2, be2, Wc, bc):
    inv = 1.0 / np.sqrt(1.0 + 1e-5)
    s1 = inv * jnp.repeat(g1, F1)
    w1f = W1.reshape(CF, C0 * F0).T * s1[None, :]
    b1f = (b1 * s1 + jnp.repeat(be1, F1))[None, :]
    s2 = inv * jnp.repeat(g2, F2)
    w2f = W2.reshape(C2 * F2, CF).T * s2[None, :]
    b2f = (b2 * s2 + jnp.repeat(be2, F2))[None, :]
    wct = Wc.T
    bcf = bc[None, :]
    rep_np, tile_np = _rep_tile_mats()
    rep = jnp.asarray(rep_np)
    tile = jnp.asarray(tile_np)
    seg = jnp.asarray(rep_np.T)
    nsum = jnp.asarray(
        np.kron(np.eye(BLOCK_N, dtype=np.float32),
                np.ones((1, D), dtype=np.float32)))

    xr = x.reshape(N, F0)
    nr = neighbor.reshape(N, D, F0)

    grid = (N // BLOCK_N,)
    out = pl.pallas_call(
        _klgl_block,
        grid=grid,
        in_specs=[
            pl.BlockSpec((BLOCK_N, F0), lambda i: (i, 0)),
            pl.BlockSpec((BLOCK_N, D, F0), lambda i: (i, 0, 0)),
            pl.BlockSpec((C0 * F0, CF), lambda i: (0, 0)),
            pl.BlockSpec((1, CF), lambda i: (0, 0)),
            pl.BlockSpec((CF, C2 * F2), lambda i: (0, 0)),
            pl.BlockSpec((1, C2 * F2), lambda i: (0, 0)),
            pl.BlockSpec((C2 * F2, NUM_CLASS), lambda i: (0, 0)),
            pl.BlockSpec((1, NUM_CLASS), lambda i: (0, 0)),
            pl.BlockSpec((CF, CFF), lambda i: (0, 0)),
            pl.BlockSpec((CF, CFF), lambda i: (0, 0)),
            pl.BlockSpec((CFF, CF), lambda i: (0, 0)),
            pl.BlockSpec((BLOCK_N, BLOCK_N * D), lambda i: (0, 0)),
        ],
        out_specs=pl.BlockSpec((BLOCK_N, NUM_CLASS), lambda i: (i, 0)),
        out_shape=jax.ShapeDtypeStruct((N, NUM_CLASS), jnp.float32),
        scratch_shapes=[pltpu.VMEM((BLOCK_N, F0), jnp.float32),
                        pltpu.VMEM((BLOCK_N, D, F0), jnp.float32)],
    )(xr, nr, w1f, b1f, w2f, b2f, wct, bcf, rep, tile, seg, nsum)
    return out


# speedup vs baseline: 1.0663x; 1.0663x over previous
import jax
import jax.numpy as jnp
import numpy as np
from jax.experimental import pallas as pl
from jax.experimental.pallas import tpu as pltpu

N, D, C0, F0 = 10000, 16, 1, 128
C1, F1 = 4, 16
C2, F2 = 32, 1
NUM_CLASS = 40
CF = C1 * F1
CFF = C1 * F1 * F1

BLOCK_N = 200
CHUNK = 2


def _softsign(v):
    return v / (1.0 + jnp.abs(v))


def _sgnroot_pair(a):
    m = a * jax.lax.rsqrt(jnp.maximum(jnp.abs(a), 1e-30))
    return m, jnp.abs(m)


def _rep_tile_mats():
    rep = np.zeros((CF, CFF), dtype=np.float32)
    tile = np.zeros((CF, CFF), dtype=np.float32)
    for c in range(C1):
        for xx in range(F1):
            for yy in range(F1):
                j = c * F1 * F1 + xx * F1 + yy
                rep[c * F1 + xx, j] = 1.0
                tile[c * F1 + yy, j] = 1.0
    return rep, tile


def _klgl_block(x_ref, nbr_ref, w1_ref, b1_ref, w2_ref, b2_ref, wc_ref,
                bc_ref, rep_ref, tile_ref, seg_ref, nsum_ref, out_ref,
                zx_scr, zn_scr):
    B = x_ref.shape[0]
    xb = x_ref[:]
    nb = nbr_ref[:]
    sb = jnp.sum(nb, axis=1)

    for c in range(B // CHUNK):
        lo = c * CHUNK
        xc = xb[lo:lo + CHUNK]
        sc = sb[lo:lo + CHUNK]
        a = xc[:, :, None] * sc[:, None, :] + sc[:, :, None] * xc[:, None, :]
        m, r = _sgnroot_pair(a)
        rs = jax.lax.dot_general(jnp.ones((CHUNK, 1, F0), jnp.bfloat16),
                                 r.astype(jnp.bfloat16),
                                 (((2,), (1,)), ((0,), (0,))),
                                 preferred_element_type=jnp.float32) + 1e-7
        v = jnp.concatenate([nb[lo:lo + CHUNK], xc[:, None, :]], axis=1)
        z = jax.lax.dot_general(v.astype(jnp.bfloat16),
                                m.astype(jnp.bfloat16),
                                (((2,), (1,)), ((0,), (0,))),
                                preferred_element_type=jnp.float32) / rs
        zn_scr[lo:lo + CHUNK] = z[:, :D, :]
        zx_scr[lo:lo + CHUNK] = z[:, D, :]

    x1 = _softsign(jnp.dot(zx_scr[:], w1_ref[:],
                           preferred_element_type=jnp.float32) + b1_ref[:])
    hn = _softsign(jnp.dot(zn_scr[:].reshape(B * D, F0), w1_ref[:],
                           preferred_element_type=jnp.float32) + b1_ref[:])
    s2 = jnp.dot(nsum_ref[:], hn, preferred_element_type=jnp.float32)

    xs = jnp.concatenate([x1, s2], axis=0)
    reps = jnp.dot(xs, rep_ref[:], preferred_element_type=jnp.float32)
    tiles = jnp.dot(xs, tile_ref[:], preferred_element_type=jnp.float32)
    a2 = reps[:B] * tiles[B:] + reps[B:] * tiles[:B]
    m2, r2 = _sgnroot_pair(a2)
    stk = jnp.concatenate([r2, m2 * tiles[:B]], axis=0)
    sums = jnp.dot(stk, seg_ref[:], preferred_element_type=jnp.float32)
    zx2 = sums[B:] / (sums[:B] + 1e-7)

    x2 = jnp.dot(zx2, w2_ref[:],
                 preferred_element_type=jnp.float32) + b2_ref[:]
    x2 = _softsign(x2)
    out_ref[:] = jnp.dot(x2, wc_ref[:],
                         preferred_element_type=jnp.float32) + bc_ref[:]


@jax.jit
def kernel(x, neighbor, W1, b1, g1, be1, W2, b2, g2, be2, Wc, bc):
    inv = 1.0 / np.sqrt(1.0 + 1e-5)
    s1 = inv * jnp.repeat(g1, F1)
    w1f = W1.reshape(CF, C0 * F0).T * s1[None, :]
    b1f = (b1 * s1 + jnp.repeat(be1, F1))[None, :]
    s2 = inv * jnp.repeat(g2, F2)
    w2f = W2.reshape(C2 * F2, CF).T * s2[None, :]
    b2f = (b2 * s2 + jnp.repeat(be2, F2))[None, :]
    wct = Wc.T
    bcf = bc[None, :]
    rep_np, tile_np = _rep_tile_mats()
    rep = jnp.asarray(rep_np)
    tile = jnp.asarray(tile_np)
    seg = jnp.asarray(rep_np.T)
    nsum = jnp.asarray(
        np.kron(np.eye(BLOCK_N, dtype=np.float32),
                np.ones((1, D), dtype=np.float32)))

    xr = x.reshape(N, F0)
    nr = neighbor.reshape(N, D, F0)

    grid = (N // BLOCK_N,)
    out = pl.pallas_call(
        _klgl_block,
        grid=grid,
        in_specs=[
            pl.BlockSpec((BLOCK_N, F0), lambda i: (i, 0)),
            pl.BlockSpec((BLOCK_N, D, F0), lambda i: (i, 0, 0)),
            pl.BlockSpec((C0 * F0, CF), lambda i: (0, 0)),
            pl.BlockSpec((1, CF), lambda i: (0, 0)),
            pl.BlockSpec((CF, C2 * F2), lambda i: (0, 0)),
            pl.BlockSpec((1, C2 * F2), lambda i: (0, 0)),
            pl.BlockSpec((C2 * F2, NUM_CLASS), lambda i: (0, 0)),
            pl.BlockSpec((1, NUM_CLASS), lambda i: (0, 0)),
            pl.BlockSpec((CF, CFF), lambda i: (0, 0)),
            pl.BlockSpec((CF, CFF), lambda i: (0, 0)),
            pl.BlockSpec((CFF, CF), lambda i: (0, 0)),
            pl.BlockSpec((BLOCK_N, BLOCK_N * D), lambda i: (0, 0)),
        ],
        out_specs=pl.BlockSpec((BLOCK_N, NUM_CLASS), lambda i: (i, 0)),
        out_shape=jax.ShapeDtypeStruct((N, NUM_CLASS), jnp.float32),
        scratch_shapes=[pltpu.VMEM((BLOCK_N, F0), jnp.float32),
                        pltpu.VMEM((BLOCK_N, D, F0), jnp.float32)],
    )(xr, nr, w1f, b1f, w2f, b2f, wct, bcf, rep, tile, seg, nsum)
    return out


# speedup vs baseline: 1.0732x; 1.0065x over previous
import jax
import jax.numpy as jnp
import numpy as np
from jax.experimental import pallas as pl
from jax.experimental.pallas import tpu as pltpu

N, D, C0, F0 = 10000, 16, 1, 128
C1, F1 = 4, 16
C2, F2 = 32, 1
NUM_CLASS = 40
CF = C1 * F1
CFF = C1 * F1 * F1

BLOCK_N = 200
CHUNK = 1


def _softsign(v):
    return v / (1.0 + jnp.abs(v))


def _sgnroot_pair(a):
    m = a * jax.lax.rsqrt(jnp.maximum(jnp.abs(a), 1e-30))
    return m, jnp.abs(m)


def _rep_tile_mats():
    rep = np.zeros((CF, CFF), dtype=np.float32)
    tile = np.zeros((CF, CFF), dtype=np.float32)
    for c in range(C1):
        for xx in range(F1):
            for yy in range(F1):
                j = c * F1 * F1 + xx * F1 + yy
                rep[c * F1 + xx, j] = 1.0
                tile[c * F1 + yy, j] = 1.0
    return rep, tile


def _klgl_block(x_ref, nbr_ref, w1_ref, b1_ref, w2_ref, b2_ref, wc_ref,
                bc_ref, rep_ref, tile_ref, seg_ref, nsum_ref, out_ref,
                zx_scr, zn_scr):
    B = x_ref.shape[0]
    xb = x_ref[:]
    nb = nbr_ref[:]
    sb = jnp.sum(nb, axis=1)

    for c in range(B // CHUNK):
        lo = c * CHUNK
        xc = xb[lo:lo + CHUNK]
        sc = sb[lo:lo + CHUNK]
        a = xc[:, :, None] * sc[:, None, :] + sc[:, :, None] * xc[:, None, :]
        m, r = _sgnroot_pair(a)
        rs = jax.lax.dot_general(jnp.ones((CHUNK, 1, F0), jnp.float32),
                                 r, (((2,), (1,)), ((0,), (0,))),
                                 preferred_element_type=jnp.float32) + 1e-7
        v = jnp.concatenate([nb[lo:lo + CHUNK], xc[:, None, :]], axis=1)
        z = jax.lax.dot_general(v, m, (((2,), (1,)), ((0,), (0,))),
                                preferred_element_type=jnp.float32) / rs
        zn_scr[lo:lo + CHUNK] = z[:, :D, :]
        zx_scr[lo:lo + CHUNK] = z[:, D, :]

    x1 = _softsign(jnp.dot(zx_scr[:], w1_ref[:],
                           preferred_element_type=jnp.float32) + b1_ref[:])
    hn = _softsign(jnp.dot(zn_scr[:].reshape(B * D, F0), w1_ref[:],
                           preferred_element_type=jnp.float32) + b1_ref[:])
    s2 = jnp.dot(nsum_ref[:], hn, preferred_element_type=jnp.float32)

    xs = jnp.concatenate([x1, s2], axis=0)
    reps = jnp.dot(xs, rep_ref[:], preferred_element_type=jnp.float32)
    tiles = jnp.dot(xs, tile_ref[:], preferred_element_type=jnp.float32)
    a2 = reps[:B] * tiles[B:] + reps[B:] * tiles[:B]
    m2, r2 = _sgnroot_pair(a2)
    stk = jnp.concatenate([r2, m2 * tiles[:B]], axis=0)
    sums = jnp.dot(stk, seg_ref[:], preferred_element_type=jnp.float32)
    zx2 = sums[B:] / (sums[:B] + 1e-7)

    x2 = jnp.dot(zx2, w2_ref[:],
                 preferred_element_type=jnp.float32) + b2_ref[:]
    x2 = _softsign(x2)
    out_ref[:] = jnp.dot(x2, wc_ref[:],
                         preferred_element_type=jnp.float32) + bc_ref[:]


@jax.jit
def kernel(x, neighbor, W1, b1, g1, be1, W2, b2, g2, be2, Wc, bc):
    inv = 1.0 / np.sqrt(1.0 + 1e-5)
    s1 = inv * jnp.repeat(g1, F1)
    w1f = W1.reshape(CF, C0 * F0).T * s1[None, :]
    b1f = (b1 * s1 + jnp.repeat(be1, F1))[None, :]
    s2 = inv * jnp.repeat(g2, F2)
    w2f = W2.reshape(C2 * F2, CF).T * s2[None, :]
    b2f = (b2 * s2 + jnp.repeat(be2, F2))[None, :]
    wct = Wc.T
    bcf = bc[None, :]
    rep_np, tile_np = _rep_tile_mats()
    rep = jnp.asarray(rep_np)
    tile = jnp.asarray(tile_np)
    seg = jnp.asarray(rep_np.T)
    nsum = jnp.asarray(
        np.kron(np.eye(BLOCK_N, dtype=np.float32),
                np.ones((1, D), dtype=np.float32)))

    xr = x.reshape(N, F0)
    nr = neighbor.reshape(N, D, F0)

    grid = (N // BLOCK_N,)
    out = pl.pallas_call(
        _klgl_block,
        grid=grid,
        in_specs=[
            pl.BlockSpec((BLOCK_N, F0), lambda i: (i, 0)),
            pl.BlockSpec((BLOCK_N, D, F0), lambda i: (i, 0, 0)),
            pl.BlockSpec((C0 * F0, CF), lambda i: (0, 0)),
            pl.BlockSpec((1, CF), lambda i: (0, 0)),
            pl.BlockSpec((CF, C2 * F2), lambda i: (0, 0)),
            pl.BlockSpec((1, C2 * F2), lambda i: (0, 0)),
            pl.BlockSpec((C2 * F2, NUM_CLASS), lambda i: (0, 0)),
            pl.BlockSpec((1, NUM_CLASS), lambda i: (0, 0)),
            pl.BlockSpec((CF, CFF), lambda i: (0, 0)),
            pl.BlockSpec((CF, CFF), lambda i: (0, 0)),
            pl.BlockSpec((CFF, CF), lambda i: (0, 0)),
            pl.BlockSpec((BLOCK_N, BLOCK_N * D), lambda i: (0, 0)),
        ],
        out_specs=pl.BlockSpec((BLOCK_N, NUM_CLASS), lambda i: (i, 0)),
        out_shape=jax.ShapeDtypeStruct((N, NUM_CLASS), jnp.float32),
        scratch_shapes=[pltpu.VMEM((BLOCK_N, F0), jnp.float32),
                        pltpu.VMEM((BLOCK_N, D, F0), jnp.float32)],
    )(xr, nr, w1f, b1f, w2f, b2f, wct, bcf, rep, tile, seg, nsum)
    return out
